# baseline (speedup 1.0000x reference)
import functools

import jax
import jax.numpy as jnp
from jax import lax
from jax.experimental import pallas as pl
from jax.experimental.pallas import tpu as pltpu
from jax.experimental.pallas import tpu_sc as plsc

VOCAB = 100000
DIM = 128
BATCH = 4096
SEQ = 200
LPAD = 208
S0 = 112
S1 = SEQ - S0
NWORK = 32
RPT = BATCH // NWORK
VBLK = 4000
NCHUNK = LPAD // 16
NW = DIM // 4


def _prep_body(w_ref, a_ref, b_ref, s_ref, scl_ref, q_ref):
    w = w_ref[...]
    z = jnp.dot(w, a_ref[...], preferred_element_type=jnp.float32)
    s_ref[...] = jnp.tanh(z + b_ref[0])
    m = jnp.max(jnp.abs(w), axis=1, keepdims=True)
    m = jnp.maximum(m, 1e-30)
    scl_ref[...] = m * (1.0 / 127.0)
    r = 127.0 / m
    q_ref[...] = jnp.clip(jnp.rint(w * r), -127.0, 127.0).astype(jnp.int8)


def _prep(W_emb, W_att, b_att):
    return pl.pallas_call(
        _prep_body,
        grid=(VOCAB // VBLK,),
        in_specs=[
            pl.BlockSpec((VBLK, DIM), lambda i: (i, 0)),
            pl.BlockSpec((DIM, 1), lambda i: (0, 0)),
            pl.BlockSpec(memory_space=pltpu.SMEM),
        ],
        out_specs=[
            pl.BlockSpec((VBLK, 1), lambda i: (i, 0)),
            pl.BlockSpec((VBLK, 1), lambda i: (i, 0)),
            pl.BlockSpec((VBLK, DIM), lambda i: (i, 0)),
        ],
        out_shape=[
            jax.ShapeDtypeStruct((VOCAB, 1), jnp.float32),
            jax.ShapeDtypeStruct((VOCAB, 1), jnp.float32),
            jax.ShapeDtypeStruct((VOCAB, DIM), jnp.int8),
        ],
    )(W_emb, W_att, b_att)


def _sc_body(x2_hbm, s_hbm, scl_hbm, q_hbm, out_hbm,
             idx_v, rows_v, sc0_v, sc1_v, sl0_v, sl1_v, w_v, out_v,
             sem0, sem1):
    wid = lax.axis_index("s") * 2 + lax.axis_index("c")
    base = wid * RPT
    pltpu.sync_copy(x2_hbm.at[pl.ds(base * 2, 2 * RPT)], idx_v)
    sems = (sem0, sem1)
    scs = (sc0_v, sc1_v)
    sls = (sl0_v, sl1_v)

    zero = jnp.zeros((16,), jnp.float32)
    for b in range(2):
        sls[b][pl.ds(192, 16)] = zero
        for r in range(SEQ, LPAD):
            rows_v[b, r, pl.ds(0, 16)] = plsc.bitcast(zero, jnp.int32)
            rows_v[b, r, pl.ds(16, 16)] = plsc.bitcast(zero, jnp.int32)

    def fire(i, buf):
        r0 = idx_v.at[2 * i]
        r1 = idx_v.at[2 * i + 1, pl.ds(0, S1)]
        pltpu.async_copy(s_hbm.at[r0], scs[buf].at[pl.ds(0, S0)], sems[buf])
        pltpu.async_copy(s_hbm.at[r1], scs[buf].at[pl.ds(S0, S1)], sems[buf])
        pltpu.async_copy(scl_hbm.at[r0], sls[buf].at[pl.ds(0, S0)], sems[buf])
        pltpu.async_copy(scl_hbm.at[r1], sls[buf].at[pl.ds(S0, S1)],
                         sems[buf])
        pltpu.async_copy(q_hbm.at[r0], rows_v.at[buf, pl.ds(0, S0)],
                         sems[buf])
        pltpu.async_copy(q_hbm.at[r1], rows_v.at[buf, pl.ds(S0, S1)],
                         sems[buf])

    def drain(buf):
        pltpu.make_async_copy(s_hbm.at[pl.ds(0, SEQ)],
                              scs[buf].at[pl.ds(0, SEQ)], sems[buf]).wait()
        pltpu.make_async_copy(scl_hbm.at[pl.ds(0, SEQ)],
                              sls[buf].at[pl.ds(0, SEQ)], sems[buf]).wait()
        pltpu.make_async_copy(q_hbm.at[pl.ds(0, SEQ)],
                              rows_v.at[buf, pl.ds(0, SEQ)], sems[buf]).wait()

    def compute(i, buf):
        def sum_body(c, acc):
            t = scs[buf][pl.ds(c * 16, 16)]
            pos = c * 16 + lax.iota(jnp.int32, 16)
            e = jnp.where(pos < SEQ, jnp.exp(t), 0.0)
            w_v[pl.ds(c * 16, 16)] = e
            return acc + e

        acc = lax.fori_loop(0, NCHUNK, sum_body, jnp.zeros((16,), jnp.float32))
        total = acc[0]
        for j in range(1, 16):
            total = total + acc[j]
        inv = 1.0 / jnp.full((16,), total, jnp.float32)

        def chunk_body(c, accs):
            w16 = w_v[pl.ds(c * 16, 16)] * inv * sls[buf][pl.ds(c * 16, 16)]
            accs = list(accs)
            for j in range(16):
                w = w16[j]
                for g in range(2):
                    v = rows_v[buf, c * 16 + j, pl.ds(g * 16, 16)]
                    b0 = ((v << 24) >> 24).astype(jnp.float32)
                    b1 = ((v << 16) >> 24).astype(jnp.float32)
                    b2 = ((v << 8) >> 24).astype(jnp.float32)
                    b3 = (v >> 24).astype(jnp.float32)
                    accs[g * 4 + 0] = accs[g * 4 + 0] + w * b0
                    accs[g * 4 + 1] = accs[g * 4 + 1] + w * b1
                    accs[g * 4 + 2] = accs[g * 4 + 2] + w * b2
                    accs[g * 4 + 3] = accs[g * 4 + 3] + w * b3
            return tuple(accs)

        accs = lax.fori_loop(
            0, NCHUNK, chunk_body,
            tuple(jnp.zeros((16,), jnp.float32) for _ in range(8)))
        for k in range(8):
            out_v[i, pl.ds(k * 16, 16)] = accs[k]

    fire(0, 0)

    def pair_body(p, carry):
        i0 = 2 * p
        fire(i0 + 1, 1)
        drain(0)
        compute(i0, 0)
        fire(jnp.minimum(i0 + 2, RPT - 1), 0)
        drain(1)
        compute(i0 + 1, 1)
        return carry

    lax.fori_loop(0, RPT // 2, pair_body, 0)
    drain(0)
    pltpu.sync_copy(out_v, out_hbm.at[pl.ds(base, RPT)])


_sc_call = functools.partial(
    pl.kernel,
    out_type=jax.ShapeDtypeStruct((BATCH, DIM), jnp.float32),
    mesh=plsc.VectorSubcoreMesh(core_axis_name="c", subcore_axis_name="s"),
    scratch_types=[
        pltpu.VMEM((2 * RPT, S0), jnp.int32),
        pltpu.VMEM((2, LPAD, NW), jnp.int32),
        pltpu.VMEM((LPAD,), jnp.float32),
        pltpu.VMEM((LPAD,), jnp.float32),
        pltpu.VMEM((LPAD,), jnp.float32),
        pltpu.VMEM((LPAD,), jnp.float32),
        pltpu.VMEM((LPAD,), jnp.float32),
        pltpu.VMEM((RPT, DIM), jnp.float32),
        pltpu.SemaphoreType.DMA,
        pltpu.SemaphoreType.DMA,
    ],
    compiler_params=pltpu.CompilerParams(
        use_tc_tiling_on_sc=False, needs_layout_passes=False),
)


@jax.jit
def kernel(x, W_emb, W_att, b_att):
    s, scl, q = _prep(W_emb, W_att, b_att)
    qi = lax.bitcast_convert_type(q.reshape(VOCAB, NW, 4), jnp.int32)
    x = x.astype(jnp.int32)
    x2 = jnp.concatenate(
        [x, jnp.zeros((BATCH, 2 * S0 - SEQ), jnp.int32)], axis=1
    ).reshape(2 * BATCH, S0)
    out = _sc_call(_sc_body)(x2, s.reshape(VOCAB), scl.reshape(VOCAB), qi)
    return (out.reshape(BATCH, 2, 4, 16)
               .transpose(0, 1, 3, 2)
               .reshape(BATCH, DIM))


# speedup vs baseline: 1.3002x; 1.3002x over previous
import functools

import jax
import jax.numpy as jnp
from jax import lax
from jax.experimental import pallas as pl
from jax.experimental.pallas import tpu as pltpu
from jax.experimental.pallas import tpu_sc as plsc

VOCAB = 100000
DIM = 128
BATCH = 4096
SEQ = 200
LPAD = 208
S0 = 112
S1 = SEQ - S0
NWORK = 32
RPT = BATCH // NWORK
VBLK = 4000
NCHUNK = LPAD // 16
NW = DIM // 4


def _prep_body(w_ref, a_ref, b_ref, s_ref, scl_ref, q_ref):
    w = w_ref[...]
    z = jnp.dot(w, a_ref[...], preferred_element_type=jnp.float32)
    s_ref[...] = jnp.tanh(z + b_ref[0])
    m = jnp.max(jnp.abs(w), axis=1, keepdims=True)
    m = jnp.maximum(m, 1e-30)
    scl_ref[...] = m * (1.0 / 127.0)
    r = 127.0 / m
    qf = jnp.clip(jnp.rint(w * r), -127.0, 127.0)
    rowi = lax.broadcasted_iota(jnp.int32, (DIM, NW), 0)
    colj = lax.broadcasted_iota(jnp.int32, (DIM, NW), 1)
    qs = [jnp.dot(qf, (rowi == colj + 32 * k).astype(jnp.float32),
                  preferred_element_type=jnp.float32).astype(jnp.int32)
          for k in range(4)]
    q_ref[...] = ((qs[0] & 0xFF) | ((qs[1] & 0xFF) << 8)
                  | ((qs[2] & 0xFF) << 16) | (qs[3] << 24))


def _prep(W_emb, W_att, b_att):
    return pl.pallas_call(
        _prep_body,
        grid=(VOCAB // VBLK,),
        in_specs=[
            pl.BlockSpec((VBLK, DIM), lambda i: (i, 0)),
            pl.BlockSpec((DIM, 1), lambda i: (0, 0)),
            pl.BlockSpec(memory_space=pltpu.SMEM),
        ],
        out_specs=[
            pl.BlockSpec((VBLK, 1), lambda i: (i, 0)),
            pl.BlockSpec((VBLK, 1), lambda i: (i, 0)),
            pl.BlockSpec((VBLK, NW), lambda i: (i, 0)),
        ],
        out_shape=[
            jax.ShapeDtypeStruct((VOCAB, 1), jnp.float32),
            jax.ShapeDtypeStruct((VOCAB, 1), jnp.float32),
            jax.ShapeDtypeStruct((VOCAB, NW), jnp.int32),
        ],
    )(W_emb, W_att, b_att)


def _sc_body(x2_hbm, s_hbm, scl_hbm, q_hbm, out_hbm,
             idx_v, rows_v, sc0_v, sc1_v, sl0_v, sl1_v, w_v, out_v,
             sem0, sem1):
    wid = lax.axis_index("s") * 2 + lax.axis_index("c")
    base = wid * RPT
    pltpu.sync_copy(x2_hbm.at[pl.ds(base * 2, 2 * RPT)], idx_v)
    sems = (sem0, sem1)
    scs = (sc0_v, sc1_v)
    sls = (sl0_v, sl1_v)

    zero = jnp.zeros((16,), jnp.float32)
    for b in range(2):
        sls[b][pl.ds(192, 16)] = zero
        for r in range(SEQ, LPAD):
            rows_v[b, r, pl.ds(0, 16)] = plsc.bitcast(zero, jnp.int32)
            rows_v[b, r, pl.ds(16, 16)] = plsc.bitcast(zero, jnp.int32)

    def fire(i, buf):
        r0 = idx_v.at[2 * i]
        r1 = idx_v.at[2 * i + 1, pl.ds(0, S1)]
        pltpu.async_copy(s_hbm.at[r0], scs[buf].at[pl.ds(0, S0)], sems[buf])
        pltpu.async_copy(s_hbm.at[r1], scs[buf].at[pl.ds(S0, S1)], sems[buf])
        pltpu.async_copy(scl_hbm.at[r0], sls[buf].at[pl.ds(0, S0)], sems[buf])
        pltpu.async_copy(scl_hbm.at[r1], sls[buf].at[pl.ds(S0, S1)],
                         sems[buf])
        pltpu.async_copy(q_hbm.at[r0], rows_v.at[buf, pl.ds(0, S0)],
                         sems[buf])
        pltpu.async_copy(q_hbm.at[r1], rows_v.at[buf, pl.ds(S0, S1)],
                         sems[buf])

    def drain(buf):
        pltpu.make_async_copy(s_hbm.at[pl.ds(0, SEQ)],
                              scs[buf].at[pl.ds(0, SEQ)], sems[buf]).wait()
        pltpu.make_async_copy(scl_hbm.at[pl.ds(0, SEQ)],
                              sls[buf].at[pl.ds(0, SEQ)], sems[buf]).wait()
        pltpu.make_async_copy(q_hbm.at[pl.ds(0, SEQ)],
                              rows_v.at[buf, pl.ds(0, SEQ)], sems[buf]).wait()

    def compute(i, buf):
        def sum_body(c, acc):
            t = scs[buf][pl.ds(c * 16, 16)]
            pos = c * 16 + lax.iota(jnp.int32, 16)
            e = jnp.where(pos < SEQ, jnp.exp(t), 0.0)
            w_v[pl.ds(c * 16, 16)] = e
            return acc + e

        acc = lax.fori_loop(0, NCHUNK, sum_body, jnp.zeros((16,), jnp.float32))
        total = acc[0]
        for j in range(1, 16):
            total = total + acc[j]
        inv = 1.0 / jnp.full((16,), total, jnp.float32)

        def chunk_body(c, accs):
            w16 = w_v[pl.ds(c * 16, 16)] * inv * sls[buf][pl.ds(c * 16, 16)]
            accs = list(accs)
            for j in range(16):
                w = w16[j]
                for g in range(2):
                    v = rows_v[buf, c * 16 + j, pl.ds(g * 16, 16)]
                    b0 = ((v << 24) >> 24).astype(jnp.float32)
                    b1 = ((v << 16) >> 24).astype(jnp.float32)
                    b2 = ((v << 8) >> 24).astype(jnp.float32)
                    b3 = (v >> 24).astype(jnp.float32)
                    accs[g * 4 + 0] = accs[g * 4 + 0] + w * b0
                    accs[g * 4 + 1] = accs[g * 4 + 1] + w * b1
                    accs[g * 4 + 2] = accs[g * 4 + 2] + w * b2
                    accs[g * 4 + 3] = accs[g * 4 + 3] + w * b3
            return tuple(accs)

        accs = lax.fori_loop(
            0, NCHUNK, chunk_body,
            tuple(jnp.zeros((16,), jnp.float32) for _ in range(8)))
        for g in range(2):
            for b in range(4):
                out_v[i, pl.ds(32 * b + 16 * g, 16)] = accs[g * 4 + b]

    fire(0, 0)

    def pair_body(p, carry):
        i0 = 2 * p
        fire(i0 + 1, 1)
        drain(0)
        compute(i0, 0)
        fire(jnp.minimum(i0 + 2, RPT - 1), 0)
        drain(1)
        compute(i0 + 1, 1)
        return carry

    lax.fori_loop(0, RPT // 2, pair_body, 0)
    drain(0)
    pltpu.sync_copy(out_v, out_hbm.at[pl.ds(base, RPT)])


_sc_call = functools.partial(
    pl.kernel,
    out_type=jax.ShapeDtypeStruct((BATCH, DIM), jnp.float32),
    mesh=plsc.VectorSubcoreMesh(core_axis_name="c", subcore_axis_name="s"),
    scratch_types=[
        pltpu.VMEM((2 * RPT, S0), jnp.int32),
        pltpu.VMEM((2, LPAD, NW), jnp.int32),
        pltpu.VMEM((LPAD,), jnp.float32),
        pltpu.VMEM((LPAD,), jnp.float32),
        pltpu.VMEM((LPAD,), jnp.float32),
        pltpu.VMEM((LPAD,), jnp.float32),
        pltpu.VMEM((LPAD,), jnp.float32),
        pltpu.VMEM((RPT, DIM), jnp.float32),
        pltpu.SemaphoreType.DMA,
        pltpu.SemaphoreType.DMA,
    ],
    compiler_params=pltpu.CompilerParams(
        use_tc_tiling_on_sc=False, needs_layout_passes=False),
)


@jax.jit
def kernel(x, W_emb, W_att, b_att):
    s, scl, qi = _prep(W_emb, W_att, b_att)
    x = x.astype(jnp.int32)
    x2 = jnp.concatenate(
        [x, jnp.zeros((BATCH, 2 * S0 - SEQ), jnp.int32)], axis=1
    ).reshape(2 * BATCH, S0)
    return _sc_call(_sc_body)(x2, s.reshape(VOCAB), scl.reshape(VOCAB), qi)
